# initial kernel scaffold (unmeasured)
import jax
import jax.numpy as jnp
from jax import lax
from jax.experimental import pallas as pl
from jax.experimental.pallas import tpu as pltpu


def kernel(
    x,
):
    def body(*refs):
        pass

    out_shape = jax.ShapeDtypeStruct(..., jnp.float32)
    return pl.pallas_call(body, out_shape=out_shape)(...)


# baseline (device time: 194313 ns/iter reference)
import jax
import jax.numpy as jnp
from jax import lax
from jax.experimental import pallas as pl
from jax.experimental.pallas import tpu as pltpu


def kernel(x):
    m, n = x.shape

    def body(x_ref, out_ref, recv_ref, send_sems, recv_sems):
        mx = lax.axis_index("x")
        my = lax.axis_index("y")
        x_peer = (1 - mx, my)
        y_peer = (mx, 1 - my)

        barrier_sem = pltpu.get_barrier_semaphore()
        for peer in (x_peer, y_peer):
            pl.semaphore_signal(
                barrier_sem, inc=1,
                device_id=peer, device_id_type=pl.DeviceIdType.MESH,
            )
        pl.semaphore_wait(barrier_sem, 2)

        rdma1 = pltpu.make_async_remote_copy(
            src_ref=x_ref,
            dst_ref=recv_ref.at[0],
            send_sem=send_sems.at[0],
            recv_sem=recv_sems.at[0],
            device_id=x_peer,
            device_id_type=pl.DeviceIdType.MESH,
        )
        rdma1.start()
        rdma1.wait()
        out_ref[...] = x_ref[...] + recv_ref[0]

        rdma2 = pltpu.make_async_remote_copy(
            src_ref=out_ref,
            dst_ref=recv_ref.at[1],
            send_sem=send_sems.at[1],
            recv_sem=recv_sems.at[1],
            device_id=y_peer,
            device_id_type=pl.DeviceIdType.MESH,
        )
        rdma2.start()
        rdma2.wait()
        out_ref[...] = out_ref[...] + recv_ref[1]

    return pl.pallas_call(
        body,
        out_shape=jax.ShapeDtypeStruct((m, n), jnp.float32),
        in_specs=[pl.BlockSpec(memory_space=pltpu.VMEM)],
        out_specs=pl.BlockSpec(memory_space=pltpu.VMEM),
        scratch_shapes=[
            pltpu.VMEM((2, m, n), jnp.float32),
            pltpu.SemaphoreType.DMA((2,)),
            pltpu.SemaphoreType.DMA((2,)),
        ],
        compiler_params=pltpu.CompilerParams(collective_id=0),
    )(x)


# device time: 109019 ns/iter; 1.7824x vs baseline; 1.7824x over previous
import jax
import jax.numpy as jnp
from jax import lax
from jax.experimental import pallas as pl
from jax.experimental.pallas import tpu as pltpu

N_CHUNKS = 16


def kernel(x):
    m, n = x.shape
    mc = m // N_CHUNKS

    def body(x_ref, out_ref, part_ref, rx_ref, ry_ref,
             sx_sems, rx_sems, sy_sems, ry_sems):
        mx = lax.axis_index("x")
        my = lax.axis_index("y")
        x_peer = (1 - mx, my)
        y_peer = (mx, 1 - my)

        barrier_sem = pltpu.get_barrier_semaphore()
        for peer in (x_peer, y_peer):
            pl.semaphore_signal(
                barrier_sem, inc=1,
                device_id=peer, device_id_type=pl.DeviceIdType.MESH,
            )
        pl.semaphore_wait(barrier_sem, 2)

        def x_rdma(c):
            return pltpu.make_async_remote_copy(
                src_ref=x_ref.at[pl.ds(c * mc, mc)],
                dst_ref=rx_ref.at[c],
                send_sem=sx_sems.at[c],
                recv_sem=rx_sems.at[c],
                device_id=x_peer,
                device_id_type=pl.DeviceIdType.MESH,
            )

        def y_rdma(c):
            return pltpu.make_async_remote_copy(
                src_ref=part_ref.at[c],
                dst_ref=ry_ref.at[c],
                send_sem=sy_sems.at[c],
                recv_sem=ry_sems.at[c],
                device_id=y_peer,
                device_id_type=pl.DeviceIdType.MESH,
            )

        for c in range(N_CHUNKS):
            x_rdma(c).start()

        for c in range(N_CHUNKS):
            x_rdma(c).wait_recv()
            part_ref[c, :, :] = x_ref[pl.ds(c * mc, mc), :] + rx_ref[c, :, :]
            y_rdma(c).start()

        for c in range(N_CHUNKS):
            y_rdma(c).wait_recv()
            out_ref[pl.ds(c * mc, mc), :] = part_ref[c, :, :] + ry_ref[c, :, :]

        for c in range(N_CHUNKS):
            x_rdma(c).wait_send()
            y_rdma(c).wait_send()

    return pl.pallas_call(
        body,
        out_shape=jax.ShapeDtypeStruct((m, n), jnp.float32),
        in_specs=[pl.BlockSpec(memory_space=pltpu.VMEM)],
        out_specs=pl.BlockSpec(memory_space=pltpu.VMEM),
        scratch_shapes=[
            pltpu.VMEM((N_CHUNKS, mc, n), jnp.float32),
            pltpu.VMEM((N_CHUNKS, mc, n), jnp.float32),
            pltpu.VMEM((N_CHUNKS, mc, n), jnp.float32),
            pltpu.SemaphoreType.DMA((N_CHUNKS,)),
            pltpu.SemaphoreType.DMA((N_CHUNKS,)),
            pltpu.SemaphoreType.DMA((N_CHUNKS,)),
            pltpu.SemaphoreType.DMA((N_CHUNKS,)),
        ],
        compiler_params=pltpu.CompilerParams(collective_id=0),
    )(x)


# device time: 79446 ns/iter; 2.4459x vs baseline; 1.3722x over previous
import jax
import jax.numpy as jnp
from jax import lax
from jax.experimental import pallas as pl
from jax.experimental.pallas import tpu as pltpu

N_DEV = 4
STEPS = N_DEV - 1
DIRS = (1, -1, 1, -1)
B = len(DIRS)


def _coords(q):
    return (q // 2, (q % 2) ^ (q // 2))


def kernel(x):
    m, n = x.shape
    bandm = m // B
    segm = bandm // N_DEV

    def body(x_ref, out_ref, rs_tmp, ag_buf,
             rs_send, rs_recv, ag_send, ag_recv):
        mx = lax.axis_index("x")
        my = lax.axis_index("y")
        p = 2 * mx + (my ^ mx)

        def seg_off(b, s):
            return b * bandm + s * segm

        barrier_sem = pltpu.get_barrier_semaphore()
        for dq in (1, 3):
            pl.semaphore_signal(
                barrier_sem, inc=1,
                device_id=_coords(jnp.mod(p + dq, N_DEV)),
                device_id_type=pl.DeviceIdType.MESH,
            )
        pl.semaphore_wait(barrier_sem, 2)

        def rs_rdma(b, t):
            d = DIRS[b]
            send_s = jnp.mod(p - d * t, N_DEV)
            src = (x_ref if t == 0 else out_ref).at[
                pl.ds(seg_off(b, send_s), segm)]
            return pltpu.make_async_remote_copy(
                src_ref=src,
                dst_ref=rs_tmp.at[b * STEPS + t],
                send_sem=rs_send.at[b * STEPS + t],
                recv_sem=rs_recv.at[b * STEPS + t],
                device_id=_coords(jnp.mod(p + d, N_DEV)),
                device_id_type=pl.DeviceIdType.MESH,
            )

        def ag_rdma(b, t):
            d = DIRS[b]
            if t == 0:
                src = out_ref.at[pl.ds(seg_off(b, jnp.mod(p + d, N_DEV)), segm)]
            else:
                src = ag_buf.at[b * STEPS + t - 1]
            return pltpu.make_async_remote_copy(
                src_ref=src,
                dst_ref=ag_buf.at[b * STEPS + t],
                send_sem=ag_send.at[b * STEPS + t],
                recv_sem=ag_recv.at[b * STEPS + t],
                device_id=_coords(jnp.mod(p + d, N_DEV)),
                device_id_type=pl.DeviceIdType.MESH,
            )

        for b in range(B):
            rs_rdma(b, 0).start()
        for t in range(STEPS):
            for b in range(B):
                d = DIRS[b]
                rs_rdma(b, t).wait_recv()
                off = seg_off(b, jnp.mod(p - d * t - d, N_DEV))
                out_ref[pl.ds(off, segm), :] = (
                    x_ref[pl.ds(off, segm), :] + rs_tmp[b * STEPS + t])
                if t < STEPS - 1:
                    rs_rdma(b, t + 1).start()
                else:
                    for tt in range(STEPS):
                        rs_rdma(b, tt).wait_send()
                    ag_rdma(b, 0).start()

        for t in range(STEPS):
            for b in range(B):
                d = DIRS[b]
                ag_rdma(b, t).wait_recv()
                off = seg_off(b, jnp.mod(p - d * t, N_DEV))
                out_ref[pl.ds(off, segm), :] = ag_buf[b * STEPS + t]
                if t < STEPS - 1:
                    ag_rdma(b, t + 1).start()

        for b in range(B):
            for t in range(STEPS):
                ag_rdma(b, t).wait_send()

    return pl.pallas_call(
        body,
        out_shape=jax.ShapeDtypeStruct((m, n), jnp.float32),
        in_specs=[pl.BlockSpec(memory_space=pltpu.VMEM)],
        out_specs=pl.BlockSpec(memory_space=pltpu.VMEM),
        scratch_shapes=[
            pltpu.VMEM((B * STEPS, segm, n), jnp.float32),
            pltpu.VMEM((B * STEPS, segm, n), jnp.float32),
            pltpu.SemaphoreType.DMA((B * STEPS,)),
            pltpu.SemaphoreType.DMA((B * STEPS,)),
            pltpu.SemaphoreType.DMA((B * STEPS,)),
            pltpu.SemaphoreType.DMA((B * STEPS,)),
        ],
        compiler_params=pltpu.CompilerParams(collective_id=0),
    )(x)
